# Initial kernel scaffold; baseline (speedup 1.0000x reference)
import jax
import jax.numpy as jnp
from jax.experimental import pallas as pl


def kernel(x, edge_index, W1, a1s, a1d, b1, W2, a2s, a2d, b2, Wfc, bfc):
    raise NotImplementedError("write your pallas kernel here")


# speedup vs baseline: 1.0973x; 1.0973x over previous
import functools

import jax
import jax.numpy as jnp
from jax import lax
from jax.experimental import pallas as pl
from jax.experimental.pallas import tpu as pltpu
from jax.experimental.pallas import tpu_sc as plsc

N = 10000
E = 320000
D_IN = 128
HID = 64
HEADS = 8
RB = 1000


def _proj1_body(x_ref, w_ref, asd_ref, h_ref, as_ref, ad_ref):
    h = jnp.dot(x_ref[...], w_ref[...], preferred_element_type=jnp.float32)
    h_ref[...] = h
    s = jnp.dot(h, asd_ref[...], preferred_element_type=jnp.float32)
    as_ref[...] = s[:, :16]
    ad_ref[...] = s[:, 16:]


def _proj1(x, W1, asd):
    return pl.pallas_call(
        _proj1_body,
        grid=(N // RB,),
        in_specs=[
            pl.BlockSpec((RB, D_IN), lambda i: (i, 0)),
            pl.BlockSpec((D_IN, HEADS * HID), lambda i: (0, 0)),
            pl.BlockSpec((HEADS * HID, 32), lambda i: (0, 0)),
        ],
        out_specs=[
            pl.BlockSpec((RB, HEADS * HID), lambda i: (i, 0)),
            pl.BlockSpec((RB, 16), lambda i: (i, 0)),
            pl.BlockSpec((RB, 16), lambda i: (i, 0)),
        ],
        out_shape=[
            jax.ShapeDtypeStruct((N, HEADS * HID), jnp.float32),
            jax.ShapeDtypeStruct((N, 16), jnp.float32),
            jax.ShapeDtypeStruct((N, 16), jnp.float32),
        ],
    )(x, W1, asd)


def _elu(v):
    return jnp.where(v > 0, v, jnp.exp(jnp.minimum(v, 0.0)) - 1.0)


def _mid_body(o1_ref, b1_ref, w2_ref, a2_ref, h2_ref, as_ref, ad_ref):
    acc = jnp.zeros((RB, HID), jnp.float32)
    for c in range(4):
        xc = _elu(o1_ref[c] + b1_ref[0, c*128:(c+1)*128][None, :])
        acc = acc + jnp.dot(xc, w2_ref[c], preferred_element_type=jnp.float32)
    h2_ref[...] = acc
    s = jnp.dot(acc, a2_ref[...], preferred_element_type=jnp.float32)
    as_ref[...] = s[:, :16]
    ad_ref[...] = s[:, 16:]


def _mid(o1c, b1, W2c, a2comb):
    return pl.pallas_call(
        _mid_body,
        grid=(N // RB,),
        in_specs=[
            pl.BlockSpec((4, RB, 128), lambda i: (0, i, 0)),
            pl.BlockSpec((1, HEADS * HID), lambda i: (0, 0)),
            pl.BlockSpec((4, 128, HID), lambda i: (0, 0, 0)),
            pl.BlockSpec((HID, 32), lambda i: (0, 0)),
        ],
        out_specs=[
            pl.BlockSpec((RB, HID), lambda i: (i, 0)),
            pl.BlockSpec((RB, 16), lambda i: (i, 0)),
            pl.BlockSpec((RB, 16), lambda i: (i, 0)),
        ],
        out_shape=[
            jax.ShapeDtypeStruct((N, HID), jnp.float32),
            jax.ShapeDtypeStruct((N, 16), jnp.float32),
            jax.ShapeDtypeStruct((N, 16), jnp.float32),
        ],
    )(o1c, b1, W2c, a2comb)


def _fin_body(oa_ref, ob_ref, b2_ref, wfc_ref, bfc_ref, y_ref):
    t = _elu(oa_ref[...] + ob_ref[...] + b2_ref[...])
    y_ref[...] = jnp.dot(t, wfc_ref[...], preferred_element_type=jnp.float32) + bfc_ref[...]


def _fin(oa, ob, b2, wfc_p, bfc_p):
    return pl.pallas_call(
        _fin_body,
        grid=(N // RB,),
        in_specs=[
            pl.BlockSpec((RB, HID), lambda i: (i, 0)),
            pl.BlockSpec((RB, HID), lambda i: (i, 0)),
            pl.BlockSpec((1, HID), lambda i: (0, 0)),
            pl.BlockSpec((HID, 128), lambda i: (0, 0)),
            pl.BlockSpec((1, 128), lambda i: (0, 0)),
        ],
        out_specs=pl.BlockSpec((RB, 128), lambda i: (i, 0)),
        out_shape=jax.ShapeDtypeStruct((N, 128), jnp.float32),
    )(oa, ob, b2, wfc_p, bfc_p)


def _edge_layer_xla(h, as16, ad16, src, dst, heads):
    a_s = as16[:, :heads]
    a_d = ad16[:, :heads]
    e = a_s[src] + a_d[dst]
    e = jnp.where(e >= 0, e, 0.2 * e)
    ex = jnp.exp(e)
    denom = jax.ops.segment_sum(ex, dst, num_segments=N)
    alpha = ex / (denom[dst] + 1e-16)
    hh = h.reshape(N, heads, -1)
    msg = hh[src] * alpha[:, :, None]
    out = jax.ops.segment_sum(msg, dst, num_segments=N)
    return out.reshape(N, -1)


def kernel(x, edge_index, W1, a1s, a1d, b1, W2, a2s, a2d, b2, Wfc, bfc):
    src = edge_index[0]
    dst = edge_index[1]

    eye8 = jnp.eye(HEADS, dtype=jnp.float32)
    As = jnp.reshape(eye8[:, None, :] * a1s[:, :, None], (HEADS * HID, HEADS))
    Ad = jnp.reshape(eye8[:, None, :] * a1d[:, :, None], (HEADS * HID, HEADS))
    asd = jnp.concatenate([As, As, Ad, Ad], axis=1)

    h1, as16, ad16 = _proj1(x, W1, asd)

    out1 = _edge_layer_xla(h1, as16, ad16, src, dst, HEADS)

    o1c = out1.reshape(N, 4, 128).transpose(1, 0, 2)

    a2comb = jnp.concatenate(
        [jnp.repeat(a2s.T, 16, axis=1), jnp.repeat(a2d.T, 16, axis=1)], axis=1
    )
    h2, as16_2, ad16_2 = _mid(o1c, b1[None, :], W2.reshape(4, 128, HID), a2comb)

    out2 = _edge_layer_xla(h2, as16_2, ad16_2, src, dst, 1)

    wfc_p = jnp.zeros((HID, 128), jnp.float32).at[:, :2].set(Wfc)
    bfc_p = jnp.zeros((1, 128), jnp.float32).at[0, :2].set(bfc)
    y = _fin(out2, jnp.zeros_like(out2), b2[None, :], wfc_p, bfc_p)
    return y[:, :2]


# speedup vs baseline: 9.3686x; 8.5380x over previous
import functools

import jax
import jax.numpy as jnp
from jax import lax
from jax.experimental import pallas as pl
from jax.experimental.pallas import tpu as pltpu
from jax.experimental.pallas import tpu_sc as plsc

N = 10000
E = 320000
D_IN = 128
HID = 64
HEADS = 8
RB = 1000


def _proj1_body(x_ref, w_ref, asd_ref, h_ref, as_ref, ad_ref):
    h = jnp.dot(x_ref[...], w_ref[...], preferred_element_type=jnp.float32)
    h_ref[...] = h
    s = jnp.dot(h, asd_ref[...], preferred_element_type=jnp.float32)
    as_ref[...] = s[:, :16]
    ad_ref[...] = s[:, 16:]


def _proj1(x, W1, asd):
    return pl.pallas_call(
        _proj1_body,
        grid=(N // RB,),
        in_specs=[
            pl.BlockSpec((RB, D_IN), lambda i: (i, 0)),
            pl.BlockSpec((D_IN, HEADS * HID), lambda i: (0, 0)),
            pl.BlockSpec((HEADS * HID, 32), lambda i: (0, 0)),
        ],
        out_specs=[
            pl.BlockSpec((RB, HEADS * HID), lambda i: (i, 0)),
            pl.BlockSpec((RB, 16), lambda i: (i, 0)),
            pl.BlockSpec((RB, 16), lambda i: (i, 0)),
        ],
        out_shape=[
            jax.ShapeDtypeStruct((N, HEADS * HID), jnp.float32),
            jax.ShapeDtypeStruct((N, 16), jnp.float32),
            jax.ShapeDtypeStruct((N, 16), jnp.float32),
        ],
    )(x, W1, asd)


def _elu(v):
    return jnp.where(v > 0, v, jnp.exp(jnp.minimum(v, 0.0)) - 1.0)


def _mid_body(o1_ref, b1_ref, w2_ref, a2_ref, h2_ref, as_ref, ad_ref):
    acc = jnp.zeros((RB, HID), jnp.float32)
    for c in range(4):
        xc = _elu(o1_ref[c] + b1_ref[0, c*128:(c+1)*128][None, :])
        acc = acc + jnp.dot(xc, w2_ref[c], preferred_element_type=jnp.float32)
    h2_ref[...] = acc
    s = jnp.dot(acc, a2_ref[...], preferred_element_type=jnp.float32)
    as_ref[...] = s[:, :16]
    ad_ref[...] = s[:, 16:]


def _mid(o1c, b1, W2c, a2comb):
    return pl.pallas_call(
        _mid_body,
        grid=(N // RB,),
        in_specs=[
            pl.BlockSpec((4, RB, 128), lambda i: (0, i, 0)),
            pl.BlockSpec((1, HEADS * HID), lambda i: (0, 0)),
            pl.BlockSpec((4, 128, HID), lambda i: (0, 0, 0)),
            pl.BlockSpec((HID, 32), lambda i: (0, 0)),
        ],
        out_specs=[
            pl.BlockSpec((RB, HID), lambda i: (i, 0)),
            pl.BlockSpec((RB, 16), lambda i: (i, 0)),
            pl.BlockSpec((RB, 16), lambda i: (i, 0)),
        ],
        out_shape=[
            jax.ShapeDtypeStruct((N, HID), jnp.float32),
            jax.ShapeDtypeStruct((N, 16), jnp.float32),
            jax.ShapeDtypeStruct((N, 16), jnp.float32),
        ],
    )(o1c, b1, W2c, a2comb)


def _fin_body(oa_ref, ob_ref, b2_ref, wfc_ref, bfc_ref, y_ref):
    t = _elu(oa_ref[...] + ob_ref[...] + b2_ref[...])
    y_ref[...] = jnp.dot(t, wfc_ref[...], preferred_element_type=jnp.float32) + bfc_ref[...]


def _fin(oa, ob, b2, wfc_p, bfc_p):
    return pl.pallas_call(
        _fin_body,
        grid=(N // RB,),
        in_specs=[
            pl.BlockSpec((RB, HID), lambda i: (i, 0)),
            pl.BlockSpec((RB, HID), lambda i: (i, 0)),
            pl.BlockSpec((1, HID), lambda i: (0, 0)),
            pl.BlockSpec((HID, 128), lambda i: (0, 0)),
            pl.BlockSpec((1, 128), lambda i: (0, 0)),
        ],
        out_specs=pl.BlockSpec((RB, 128), lambda i: (i, 0)),
        out_shape=jax.ShapeDtypeStruct((N, 128), jnp.float32),
    )(oa, ob, b2, wfc_p, bfc_p)


SC_CORES = 2
SC_TILES = 16
NPT = N // SC_TILES


def _attn_sc(src, dst, as16, ad16):
    B = 80
    EPT1 = E // SC_TILES
    EPT2 = E // (SC_CORES * SC_TILES)
    HALF = E // SC_CORES
    mesh = plsc.VectorSubcoreMesh(core_axis_name="c", subcore_axis_name="s")

    @functools.partial(
        pl.kernel,
        out_type=jax.ShapeDtypeStruct((E, 16), jnp.float32),
        mesh=mesh,
        scratch_types=[
            pltpu.VMEM((B,), jnp.int32),
            pltpu.VMEM((B,), jnp.int32),
            pltpu.VMEM((B, 16), jnp.float32),
            pltpu.VMEM((B, 16), jnp.float32),
            pltpu.VMEM((B, 16), jnp.float32),
            pltpu.VMEM((NPT, 16), jnp.float32),
            pltpu.VMEM_SHARED((N, 16), jnp.float32),
            pltpu.SemaphoreType.DMA,
        ],
        compiler_params=pltpu.CompilerParams(use_tc_tiling_on_sc=False),
    )
    def k(src_h, dst_h, as_h, ad_h, alpha_h, src_v, dst_v, as_v, ad_v, ex_v,
          z_v, den_sh, sem):
        c = lax.axis_index("c")
        s = lax.axis_index("s")

        def zrow(j, _):
            z_v[j, :] = jnp.zeros((16,), jnp.float32)
            return 0
        lax.fori_loop(0, NPT, zrow, 0)
        pltpu.sync_copy(z_v, den_sh.at[pl.ds(s * NPT, NPT)])
        plsc.subcore_barrier()

        def p1(i, _):
            off = s * EPT1 + i * B
            pltpu.sync_copy(src_h.at[pl.ds(off, B)], src_v)
            pltpu.sync_copy(dst_h.at[pl.ds(off, B)], dst_v)
            pltpu.async_copy(as_h.at[src_v], as_v, sem).wait()
            pltpu.async_copy(ad_h.at[dst_v], ad_v, sem).wait()

            def comp(j, _):
                e = as_v[j, :] + ad_v[j, :]
                e = jnp.maximum(e, 0.2 * e)
                ex_v[j, :] = jnp.exp(e)
                return 0
            lax.fori_loop(0, B, comp, 0)
            pltpu.sync_copy(ex_v, den_sh.at[dst_v], add=True)

            @pl.when(jnp.logical_and(off >= c * HALF, off < (c + 1) * HALF))
            def _():
                pltpu.sync_copy(ex_v, alpha_h.at[pl.ds(off, B)])
            return 0
        lax.fori_loop(0, EPT1 // B, p1, 0)
        plsc.subcore_barrier()

        def p2(i, _):
            off = (c * SC_TILES + s) * EPT2 + i * B
            pltpu.sync_copy(dst_h.at[pl.ds(off, B)], dst_v)
            pltpu.sync_copy(alpha_h.at[pl.ds(off, B)], ex_v)
            pltpu.async_copy(den_sh.at[dst_v], ad_v, sem).wait()

            def comp(j, _):
                ex_v[j, :] = ex_v[j, :] / (ad_v[j, :] + 1e-16)
                return 0
            lax.fori_loop(0, B, comp, 0)
            pltpu.sync_copy(ex_v, alpha_h.at[pl.ds(off, B)])
            return 0
        lax.fori_loop(0, EPT2 // B, p2, 0)

    return k(src, dst, as16, ad16)


def _msg_sc(table, alpha, src, dst, n_chunks, F):
    B = 80
    half = n_chunks == 1
    ch_per_sc = n_chunks // SC_CORES if not half else 1
    out_rows = n_chunks if not half else SC_CORES
    ept = E // (SC_CORES * SC_TILES) if half else E // SC_TILES
    mesh = plsc.VectorSubcoreMesh(core_axis_name="c", subcore_axis_name="s")

    @functools.partial(
        pl.kernel,
        out_type=jax.ShapeDtypeStruct((out_rows * N, F), jnp.float32),
        mesh=mesh,
        scratch_types=[
            pltpu.VMEM((B,), jnp.int32),
            pltpu.VMEM((B,), jnp.int32),
            pltpu.VMEM((B, 16), jnp.float32),
            pltpu.VMEM((B, F), jnp.float32),
            pltpu.VMEM((B, F), jnp.float32),
            pltpu.VMEM((125, F), jnp.float32),
            pltpu.VMEM_SHARED((N, F), jnp.float32),
            pltpu.SemaphoreType.DMA,
        ],
        compiler_params=pltpu.CompilerParams(
            use_tc_tiling_on_sc=False, needs_layout_passes=False),
    )
    def k(tab_h, al_h, src_h, dst_h, out_h, src_v, dst_v, al_v, h_v, msg_v,
          z_v, out_sh, sem):
        c = lax.axis_index("c")
        s = lax.axis_index("s")

        def zrow(j, _):
            for fb in range(F // 16):
                z_v[j, pl.ds(fb * 16, 16)] = jnp.zeros((16,), jnp.float32)
            return 0
        lax.fori_loop(0, 125, zrow, 0)

        for q_l in range(ch_per_sc):
            q = c * ch_per_sc + q_l
            for r in range(NPT // 125):
                pltpu.sync_copy(z_v, out_sh.at[pl.ds(s * NPT + r * 125, 125)])
            plsc.subcore_barrier()

            def eloop(i, _):
                base = (c * SC_TILES + s) * ept if half else s * ept
                off = base + i * B
                pltpu.sync_copy(src_h.at[pl.ds(off, B)], src_v)
                pltpu.sync_copy(dst_h.at[pl.ds(off, B)], dst_v)
                if n_chunks > 1:
                    def sidx(j, _):
                        sl = pl.ds(j * 16, 16)
                        src_v[sl] = src_v[sl] * n_chunks + q
                        return 0
                    lax.fori_loop(0, B // 16, sidx, 0)
                pltpu.async_copy(tab_h.at[src_v], h_v, sem).wait()
                pltpu.sync_copy(al_h.at[pl.ds(off, B)], al_v)

                def comp(j, _):
                    jj = jnp.full((16,), j, jnp.int32)
                    if n_chunks > 1:
                        m0 = plsc.load_gather(
                            al_v, [jj, jnp.full((16,), 2 * q, jnp.int32)])
                        m1 = plsc.load_gather(
                            al_v, [jj, jnp.full((16,), 2 * q + 1, jnp.int32)])
                    else:
                        m0 = plsc.load_gather(
                            al_v, [jj, jnp.full((16,), 0, jnp.int32)])
                        m1 = m0
                    for fb in range(F // 16):
                        sl = pl.ds(fb * 16, 16)
                        m = m0 if fb < (F // 32) else m1
                        msg_v[j, sl] = h_v[j, sl] * m
                    return 0
                lax.fori_loop(0, B, comp, 0)
                pltpu.sync_copy(msg_v, out_sh.at[dst_v], add=True)
                return 0
            lax.fori_loop(0, ept // B, eloop, 0)
            plsc.subcore_barrier()
            ob = (q if not half else c) * N + s * NPT
            pltpu.sync_copy(out_sh.at[pl.ds(s * NPT, NPT)],
                            out_h.at[pl.ds(ob, NPT)])

    return k(table, alpha, src, dst)


def kernel(x, edge_index, W1, a1s, a1d, b1, W2, a2s, a2d, b2, Wfc, bfc):
    src = edge_index[0]
    dst = edge_index[1]

    eye8 = jnp.eye(HEADS, dtype=jnp.float32)
    As = jnp.reshape(eye8[:, None, :] * a1s[:, :, None], (HEADS * HID, HEADS))
    Ad = jnp.reshape(eye8[:, None, :] * a1d[:, :, None], (HEADS * HID, HEADS))
    asd = jnp.concatenate([As, As, Ad, Ad], axis=1)

    h1, as16, ad16 = _proj1(x, W1, asd)

    alpha1 = _attn_sc(src, dst, as16, ad16)
    out1f = _msg_sc(h1.reshape(4 * N, 128), alpha1, src, dst, 4, 128)
    o1c = out1f.reshape(4, N, 128)

    a2comb = jnp.concatenate(
        [jnp.repeat(a2s.T, 16, axis=1), jnp.repeat(a2d.T, 16, axis=1)], axis=1
    )
    h2, as16_2, ad16_2 = _mid(o1c, b1[None, :], W2.reshape(4, 128, HID), a2comb)

    alpha2 = _attn_sc(src, dst, as16_2, ad16_2)
    out2f = _msg_sc(h2, alpha2, src, dst, 1, HID)
    o2 = out2f.reshape(2, N, HID)

    wfc_p = jnp.zeros((HID, 128), jnp.float32).at[:, :2].set(Wfc)
    bfc_p = jnp.zeros((1, 128), jnp.float32).at[0, :2].set(bfc)
    y = _fin(o2[0], o2[1], b2[None, :], wfc_p, bfc_p)
    return y[:, :2]


# speedup vs baseline: 14.2399x; 1.5200x over previous
import functools

import jax
import jax.numpy as jnp
from jax import lax
from jax.experimental import pallas as pl
from jax.experimental.pallas import tpu as pltpu
from jax.experimental.pallas import tpu_sc as plsc

N = 10000
E = 320000
D_IN = 128
HID = 64
HEADS = 8
RB = 1000


def _proj1_body(x_ref, w_ref, asd_ref, h_ref, as_ref, ad_ref):
    h = jnp.dot(x_ref[...], w_ref[...], preferred_element_type=jnp.float32)
    h_ref[...] = h
    s = jnp.dot(h, asd_ref[...], preferred_element_type=jnp.float32)
    as_ref[...] = s[:, :16]
    ad_ref[...] = s[:, 16:]


def _proj1(x, W1, asd):
    return pl.pallas_call(
        _proj1_body,
        grid=(N // RB,),
        in_specs=[
            pl.BlockSpec((RB, D_IN), lambda i: (i, 0)),
            pl.BlockSpec((D_IN, HEADS * HID), lambda i: (0, 0)),
            pl.BlockSpec((HEADS * HID, 32), lambda i: (0, 0)),
        ],
        out_specs=[
            pl.BlockSpec((RB, HEADS * HID), lambda i: (i, 0)),
            pl.BlockSpec((RB, 16), lambda i: (i, 0)),
            pl.BlockSpec((RB, 16), lambda i: (i, 0)),
        ],
        out_shape=[
            jax.ShapeDtypeStruct((N, HEADS * HID), jnp.float32),
            jax.ShapeDtypeStruct((N, 16), jnp.float32),
            jax.ShapeDtypeStruct((N, 16), jnp.float32),
        ],
    )(x, W1, asd)


def _elu(v):
    return jnp.where(v > 0, v, jnp.exp(jnp.minimum(v, 0.0)) - 1.0)


def _mid_body(o1_ref, b1_ref, w2_ref, a2_ref, h2_ref, as_ref, ad_ref):
    acc = jnp.zeros((RB, HID), jnp.float32)
    for c in range(4):
        xc = _elu(o1_ref[c] + b1_ref[0, c*128:(c+1)*128][None, :])
        acc = acc + jnp.dot(xc, w2_ref[c], preferred_element_type=jnp.float32)
    h2_ref[...] = acc
    s = jnp.dot(acc, a2_ref[...], preferred_element_type=jnp.float32)
    as_ref[...] = s[:, :16]
    ad_ref[...] = s[:, 16:]


def _mid(o1c, b1, W2c, a2comb):
    return pl.pallas_call(
        _mid_body,
        grid=(N // RB,),
        in_specs=[
            pl.BlockSpec((4, RB, 128), lambda i: (0, i, 0)),
            pl.BlockSpec((1, HEADS * HID), lambda i: (0, 0)),
            pl.BlockSpec((4, 128, HID), lambda i: (0, 0, 0)),
            pl.BlockSpec((HID, 32), lambda i: (0, 0)),
        ],
        out_specs=[
            pl.BlockSpec((RB, HID), lambda i: (i, 0)),
            pl.BlockSpec((RB, 16), lambda i: (i, 0)),
            pl.BlockSpec((RB, 16), lambda i: (i, 0)),
        ],
        out_shape=[
            jax.ShapeDtypeStruct((N, HID), jnp.float32),
            jax.ShapeDtypeStruct((N, 16), jnp.float32),
            jax.ShapeDtypeStruct((N, 16), jnp.float32),
        ],
    )(o1c, b1, W2c, a2comb)


def _fin_body(oa_ref, ob_ref, b2_ref, wfc_ref, bfc_ref, y_ref):
    t = _elu(oa_ref[...] + ob_ref[...] + b2_ref[...])
    y_ref[...] = jnp.dot(t, wfc_ref[...], preferred_element_type=jnp.float32) + bfc_ref[...]


def _fin(oa, ob, b2, wfc_p, bfc_p):
    return pl.pallas_call(
        _fin_body,
        grid=(N // RB,),
        in_specs=[
            pl.BlockSpec((RB, HID), lambda i: (i, 0)),
            pl.BlockSpec((RB, HID), lambda i: (i, 0)),
            pl.BlockSpec((1, HID), lambda i: (0, 0)),
            pl.BlockSpec((HID, 128), lambda i: (0, 0)),
            pl.BlockSpec((1, 128), lambda i: (0, 0)),
        ],
        out_specs=pl.BlockSpec((RB, 128), lambda i: (i, 0)),
        out_shape=jax.ShapeDtypeStruct((N, 128), jnp.float32),
    )(oa, ob, b2, wfc_p, bfc_p)


SC_CORES = 2
SC_TILES = 16
NPT = N // SC_TILES


def _attn_sc(src, dst, as16, ad16):
    B = 1000
    EPT1 = E // SC_TILES
    EPT2 = E // (SC_CORES * SC_TILES)
    HALF = E // SC_CORES
    mesh = plsc.VectorSubcoreMesh(core_axis_name="c", subcore_axis_name="s")

    @functools.partial(
        pl.kernel,
        out_type=jax.ShapeDtypeStruct((E, 16), jnp.float32),
        mesh=mesh,
        scratch_types=[
            pltpu.VMEM((B,), jnp.int32),
            pltpu.VMEM((B,), jnp.int32),
            pltpu.VMEM((B, 16), jnp.float32),
            pltpu.VMEM((B, 16), jnp.float32),
            pltpu.VMEM((B, 16), jnp.float32),
            pltpu.VMEM((NPT, 16), jnp.float32),
            pltpu.VMEM_SHARED((N, 16), jnp.float32),
            pltpu.SemaphoreType.DMA,
        ],
        compiler_params=pltpu.CompilerParams(use_tc_tiling_on_sc=False),
    )
    def k(src_h, dst_h, as_h, ad_h, alpha_h, src_v, dst_v, as_v, ad_v, ex_v,
          z_v, den_sh, sem):
        c = lax.axis_index("c")
        s = lax.axis_index("s")

        def zrow(j, _):
            z_v[j, :] = jnp.zeros((16,), jnp.float32)
            return 0
        lax.fori_loop(0, NPT, zrow, 0)
        pltpu.sync_copy(z_v, den_sh.at[pl.ds(s * NPT, NPT)])
        plsc.subcore_barrier()

        def p1(i, _):
            off = s * EPT1 + i * B
            pltpu.sync_copy(src_h.at[pl.ds(off, B)], src_v)
            pltpu.sync_copy(dst_h.at[pl.ds(off, B)], dst_v)
            pltpu.async_copy(as_h.at[src_v], as_v, sem).wait()
            pltpu.async_copy(ad_h.at[dst_v], ad_v, sem).wait()

            def comp(j, _):
                e = as_v[j, :] + ad_v[j, :]
                e = jnp.maximum(e, 0.2 * e)
                ex_v[j, :] = jnp.exp(e)
                return 0
            lax.fori_loop(0, B, comp, 0)
            pltpu.sync_copy(ex_v, den_sh.at[dst_v], add=True)

            @pl.when(jnp.logical_and(off >= c * HALF, off < (c + 1) * HALF))
            def _():
                pltpu.sync_copy(ex_v, alpha_h.at[pl.ds(off, B)])
            return 0
        lax.fori_loop(0, EPT1 // B, p1, 0)
        plsc.subcore_barrier()

        def p2(i, _):
            off = (c * SC_TILES + s) * EPT2 + i * B
            pltpu.sync_copy(dst_h.at[pl.ds(off, B)], dst_v)
            pltpu.sync_copy(alpha_h.at[pl.ds(off, B)], ex_v)
            pltpu.async_copy(den_sh.at[dst_v], ad_v, sem).wait()

            def comp(j, _):
                ex_v[j, :] = ex_v[j, :] / (ad_v[j, :] + 1e-16)
                return 0
            lax.fori_loop(0, B, comp, 0)
            pltpu.sync_copy(ex_v, alpha_h.at[pl.ds(off, B)])
            return 0
        lax.fori_loop(0, EPT2 // B, p2, 0)

    return k(src, dst, as16, ad16)


def _msg_sc(table, alpha, src, dst, n_chunks, F, B):
    half = n_chunks == 1
    ch_per_sc = n_chunks // SC_CORES if not half else 1
    out_rows = n_chunks if not half else SC_CORES
    ept = E // (SC_CORES * SC_TILES) if half else E // SC_TILES
    mesh = plsc.VectorSubcoreMesh(core_axis_name="c", subcore_axis_name="s")

    @functools.partial(
        pl.kernel,
        out_type=jax.ShapeDtypeStruct((out_rows * N, F), jnp.float32),
        mesh=mesh,
        scratch_types=[
            pltpu.VMEM((B,), jnp.int32),
            pltpu.VMEM((B,), jnp.int32),
            pltpu.VMEM((B, 16), jnp.float32),
            pltpu.VMEM((B, F), jnp.float32),
            pltpu.VMEM((B, F), jnp.float32),
            pltpu.VMEM_SHARED((N, F), jnp.float32),
            pltpu.SemaphoreType.DMA,
        ],
        compiler_params=pltpu.CompilerParams(
            use_tc_tiling_on_sc=False, needs_layout_passes=False),
    )
    def k(tab_h, al_h, src_h, dst_h, out_h, src_v, dst_v, al_v, h_v, msg_v,
          out_sh, sem):
        c = lax.axis_index("c")
        s = lax.axis_index("s")

        for q_l in range(ch_per_sc):
            q = c * ch_per_sc + q_l

            def zrow(j, _):
                for fb in range(F // 16):
                    msg_v[j, pl.ds(fb * 16, 16)] = jnp.zeros((16,), jnp.float32)
                return 0
            lax.fori_loop(0, 125, zrow, 0)
            for r in range(NPT // 125):
                pltpu.sync_copy(msg_v.at[pl.ds(0, 125)],
                                out_sh.at[pl.ds(s * NPT + r * 125, 125)])
            plsc.subcore_barrier()

            def eloop(i, _):
                base = (c * SC_TILES + s) * ept if half else s * ept
                off = base + i * B
                pltpu.sync_copy(src_h.at[pl.ds(off, B)], src_v)
                pltpu.sync_copy(dst_h.at[pl.ds(off, B)], dst_v)
                if n_chunks > 1:
                    def sidx(j, _):
                        sl = pl.ds(j * 16, 16)
                        src_v[sl] = src_v[sl] * n_chunks + q
                        return 0
                    lax.fori_loop(0, B // 16, sidx, 0)
                pltpu.async_copy(tab_h.at[src_v], h_v, sem).wait()
                pltpu.sync_copy(al_h.at[pl.ds(off, B)], al_v)

                def comp(j, _):
                    jj = jnp.full((16,), j, jnp.int32)
                    if n_chunks > 1:
                        m0 = plsc.load_gather(
                            al_v, [jj, jnp.full((16,), 2 * q, jnp.int32)])
                        m1 = plsc.load_gather(
                            al_v, [jj, jnp.full((16,), 2 * q + 1, jnp.int32)])
                    else:
                        m0 = plsc.load_gather(
                            al_v, [jj, jnp.full((16,), 0, jnp.int32)])
                        m1 = m0
                    for fb in range(F // 16):
                        sl = pl.ds(fb * 16, 16)
                        m = m0 if fb < (F // 32) else m1
                        msg_v[j, sl] = h_v[j, sl] * m
                    return 0
                lax.fori_loop(0, B, comp, 0)
                pltpu.sync_copy(msg_v, out_sh.at[dst_v], add=True)
                return 0
            lax.fori_loop(0, ept // B, eloop, 0)
            plsc.subcore_barrier()
            ob = (q if not half else c) * N + s * NPT
            pltpu.sync_copy(out_sh.at[pl.ds(s * NPT, NPT)],
                            out_h.at[pl.ds(ob, NPT)])

    return k(table, alpha, src, dst)


def kernel(x, edge_index, W1, a1s, a1d, b1, W2, a2s, a2d, b2, Wfc, bfc):
    src = edge_index[0]
    dst = edge_index[1]

    eye8 = jnp.eye(HEADS, dtype=jnp.float32)
    As = jnp.reshape(eye8[:, None, :] * a1s[:, :, None], (HEADS * HID, HEADS))
    Ad = jnp.reshape(eye8[:, None, :] * a1d[:, :, None], (HEADS * HID, HEADS))
    asd = jnp.concatenate([As, As, Ad, Ad], axis=1)

    h1, as16, ad16 = _proj1(x, W1, asd)

    alpha1 = _attn_sc(src, dst, as16, ad16)
    out1f = _msg_sc(h1.reshape(4 * N, 128), alpha1, src, dst, 4, 128, 160)
    o1c = out1f.reshape(4, N, 128)

    a2comb = jnp.concatenate(
        [jnp.repeat(a2s.T, 16, axis=1), jnp.repeat(a2d.T, 16, axis=1)], axis=1
    )
    h2, as16_2, ad16_2 = _mid(o1c, b1[None, :], W2.reshape(4, 128, HID), a2comb)

    alpha2 = _attn_sc(src, dst, as16_2, ad16_2)
    out2f = _msg_sc(h2, alpha2, src, dst, 1, HID, 200)
    o2 = out2f.reshape(2, N, HID)

    wfc_p = jnp.zeros((HID, 128), jnp.float32).at[:, :2].set(Wfc)
    bfc_p = jnp.zeros((1, 128), jnp.float32).at[0, :2].set(bfc)
    y = _fin(o2[0], o2[1], b2[None, :], wfc_p, bfc_p)
    return y[:, :2]


# speedup vs baseline: 26.0289x; 1.8279x over previous
import functools

import jax
import jax.numpy as jnp
from jax import lax
from jax.experimental import pallas as pl
from jax.experimental.pallas import tpu as pltpu
from jax.experimental.pallas import tpu_sc as plsc

N = 10000
E = 320000
D_IN = 128
HID = 64
HEADS = 8
RB = 1000


def _proj1_body(x_ref, w_ref, asd_ref, h_ref, as_ref, ad_ref):
    h = jnp.dot(x_ref[...], w_ref[...], preferred_element_type=jnp.float32)
    h_ref[...] = h
    s = jnp.dot(h, asd_ref[...], preferred_element_type=jnp.float32)
    as_ref[...] = s[:, :16]
    ad_ref[...] = s[:, 16:]


def _proj1(x, W1, asd):
    return pl.pallas_call(
        _proj1_body,
        grid=(N // RB,),
        in_specs=[
            pl.BlockSpec((RB, D_IN), lambda i: (i, 0)),
            pl.BlockSpec((D_IN, HEADS * HID), lambda i: (0, 0)),
            pl.BlockSpec((HEADS * HID, 32), lambda i: (0, 0)),
        ],
        out_specs=[
            pl.BlockSpec((RB, HEADS * HID), lambda i: (i, 0)),
            pl.BlockSpec((RB, 16), lambda i: (i, 0)),
            pl.BlockSpec((RB, 16), lambda i: (i, 0)),
        ],
        out_shape=[
            jax.ShapeDtypeStruct((N, HEADS * HID), jnp.float32),
            jax.ShapeDtypeStruct((N, 16), jnp.float32),
            jax.ShapeDtypeStruct((N, 16), jnp.float32),
        ],
    )(x, W1, asd)


def _elu(v):
    return jnp.where(v > 0, v, jnp.exp(jnp.minimum(v, 0.0)) - 1.0)


def _mid_body(o1_ref, b1_ref, w2_ref, a2_ref, h2_ref, as_ref, ad_ref):
    acc = jnp.zeros((RB, HID), jnp.float32)
    for c in range(4):
        xc = _elu(o1_ref[c] + b1_ref[0, c*128:(c+1)*128][None, :])
        acc = acc + jnp.dot(xc, w2_ref[c], preferred_element_type=jnp.float32)
    h2_ref[...] = acc
    s = jnp.dot(acc, a2_ref[...], preferred_element_type=jnp.float32)
    as_ref[...] = s[:, :16]
    ad_ref[...] = s[:, 16:]


def _mid(o1c, b1, W2c, a2comb):
    return pl.pallas_call(
        _mid_body,
        grid=(N // RB,),
        in_specs=[
            pl.BlockSpec((4, RB, 128), lambda i: (0, i, 0)),
            pl.BlockSpec((1, HEADS * HID), lambda i: (0, 0)),
            pl.BlockSpec((4, 128, HID), lambda i: (0, 0, 0)),
            pl.BlockSpec((HID, 32), lambda i: (0, 0)),
        ],
        out_specs=[
            pl.BlockSpec((RB, HID), lambda i: (i, 0)),
            pl.BlockSpec((RB, 16), lambda i: (i, 0)),
            pl.BlockSpec((RB, 16), lambda i: (i, 0)),
        ],
        out_shape=[
            jax.ShapeDtypeStruct((N, HID), jnp.float32),
            jax.ShapeDtypeStruct((N, 16), jnp.float32),
            jax.ShapeDtypeStruct((N, 16), jnp.float32),
        ],
    )(o1c, b1, W2c, a2comb)


def _fin_body(oa_ref, ob_ref, b2_ref, wfc_ref, bfc_ref, y_ref):
    t = _elu(oa_ref[...] + ob_ref[...] + b2_ref[...])
    y_ref[...] = jnp.dot(t, wfc_ref[...], preferred_element_type=jnp.float32) + bfc_ref[...]


def _fin(oa, ob, b2, wfc_p, bfc_p):
    return pl.pallas_call(
        _fin_body,
        grid=(N // RB,),
        in_specs=[
            pl.BlockSpec((RB, HID), lambda i: (i, 0)),
            pl.BlockSpec((RB, HID), lambda i: (i, 0)),
            pl.BlockSpec((1, HID), lambda i: (0, 0)),
            pl.BlockSpec((HID, 128), lambda i: (0, 0)),
            pl.BlockSpec((1, 128), lambda i: (0, 0)),
        ],
        out_specs=pl.BlockSpec((RB, 128), lambda i: (i, 0)),
        out_shape=jax.ShapeDtypeStruct((N, 128), jnp.float32),
    )(oa, ob, b2, wfc_p, bfc_p)


SC_CORES = 2
SC_TILES = 16
NPT = N // SC_TILES


def _attn_sc(src, dst, as16, ad16):
    B = 1000
    EPT1 = E // SC_TILES
    EPT2 = E // (SC_CORES * SC_TILES)
    HALF = E // SC_CORES
    mesh = plsc.VectorSubcoreMesh(core_axis_name="c", subcore_axis_name="s")

    @functools.partial(
        pl.kernel,
        out_type=jax.ShapeDtypeStruct((E, 16), jnp.float32),
        mesh=mesh,
        scratch_types=[
            pltpu.VMEM((B,), jnp.int32),
            pltpu.VMEM((B,), jnp.int32),
            pltpu.VMEM((B, 16), jnp.float32),
            pltpu.VMEM((B, 16), jnp.float32),
            pltpu.VMEM((B, 16), jnp.float32),
            pltpu.VMEM((NPT, 16), jnp.float32),
            pltpu.VMEM_SHARED((N, 16), jnp.float32),
            pltpu.SemaphoreType.DMA,
        ],
        compiler_params=pltpu.CompilerParams(use_tc_tiling_on_sc=False),
    )
    def k(src_h, dst_h, as_h, ad_h, alpha_h, src_v, dst_v, as_v, ad_v, ex_v,
          z_v, den_sh, sem):
        c = lax.axis_index("c")
        s = lax.axis_index("s")

        def zrow(j, _):
            z_v[j, :] = jnp.zeros((16,), jnp.float32)
            return 0
        lax.fori_loop(0, NPT, zrow, 0)
        pltpu.sync_copy(z_v, den_sh.at[pl.ds(s * NPT, NPT)])
        plsc.subcore_barrier()

        def p1(i, _):
            off = s * EPT1 + i * B
            pltpu.sync_copy(src_h.at[pl.ds(off, B)], src_v)
            pltpu.sync_copy(dst_h.at[pl.ds(off, B)], dst_v)
            pltpu.async_copy(as_h.at[src_v], as_v, sem).wait()
            pltpu.async_copy(ad_h.at[dst_v], ad_v, sem).wait()

            def comp(j, _):
                e = as_v[j, :] + ad_v[j, :]
                e = jnp.maximum(e, 0.2 * e)
                ex_v[j, :] = jnp.exp(e)
                return 0
            lax.fori_loop(0, B, comp, 0)
            pltpu.sync_copy(ex_v, den_sh.at[dst_v], add=True)

            @pl.when(jnp.logical_and(off >= c * HALF, off < (c + 1) * HALF))
            def _():
                pltpu.sync_copy(ex_v, alpha_h.at[pl.ds(off, B)])
            return 0
        lax.fori_loop(0, EPT1 // B, p1, 0)
        plsc.subcore_barrier()

        def p2(i, _):
            off = (c * SC_TILES + s) * EPT2 + i * B
            pltpu.sync_copy(dst_h.at[pl.ds(off, B)], dst_v)
            pltpu.sync_copy(alpha_h.at[pl.ds(off, B)], ex_v)
            pltpu.async_copy(den_sh.at[dst_v], ad_v, sem).wait()

            def comp(j, _):
                ex_v[j, :] = ex_v[j, :] / (ad_v[j, :] + 1e-16)
                return 0
            lax.fori_loop(0, B, comp, 0)
            pltpu.sync_copy(ex_v, alpha_h.at[pl.ds(off, B)])
            return 0
        lax.fori_loop(0, EPT2 // B, p2, 0)

    return k(src, dst, as16, ad16)


def _msg_sc(table, alpha, src, dst, n_chunks, F, B):
    half = n_chunks == 1
    ch_per_sc = n_chunks // SC_CORES if not half else 1
    out_rows = n_chunks if not half else SC_CORES
    ept = E // (SC_CORES * SC_TILES) if half else E // SC_TILES
    mesh = plsc.VectorSubcoreMesh(core_axis_name="c", subcore_axis_name="s")

    niter = ept // B
    assert niter * B == ept and B % 4 == 0
    assert n_chunks == 1 or B % 16 == 0

    @functools.partial(
        pl.kernel,
        out_type=jax.ShapeDtypeStruct((out_rows * N, F), jnp.float32),
        mesh=mesh,
        scratch_types=[
            pltpu.VMEM((B,), jnp.int32),
            pltpu.VMEM((B,), jnp.int32),
            pltpu.VMEM((B,), jnp.int32),
            pltpu.VMEM((B,), jnp.int32),
            pltpu.VMEM((B, 16), jnp.float32),
            pltpu.VMEM((B, 16), jnp.float32),
            pltpu.VMEM((B, F), jnp.float32),
            pltpu.VMEM((B, F), jnp.float32),
            pltpu.VMEM_SHARED((N, F), jnp.float32),
            pltpu.SemaphoreType.DMA,
            pltpu.SemaphoreType.DMA,
        ],
        compiler_params=pltpu.CompilerParams(
            use_tc_tiling_on_sc=False, needs_layout_passes=False),
    )
    def k(tab_h, al_h, src_h, dst_h, out_h, src0, src1, dst0, dst1, al0, al1,
          h0, h1, out_sh, sem0, sem1):
        c = lax.axis_index("c")
        s = lax.axis_index("s")
        srcs, dsts, als, hs, sems = (src0, src1), (dst0, dst1), (al0, al1), \
            (h0, h1), (sem0, sem1)

        for q_l in range(ch_per_sc):
            q = c * ch_per_sc + q_l
            base = (c * SC_TILES + s) * ept if half else s * ept

            def zrow(j, _):
                for fb in range(F // 16):
                    h0[j, pl.ds(fb * 16, 16)] = jnp.zeros((16,), jnp.float32)
                return 0
            lax.fori_loop(0, 125, zrow, 0)
            for r in range(NPT // 125):
                pltpu.sync_copy(h0.at[pl.ds(0, 125)],
                                out_sh.at[pl.ds(s * NPT + r * 125, 125)])
            plsc.subcore_barrier()

            def prefetch(i, sl):
                off = base + i * B
                pltpu.sync_copy(src_h.at[pl.ds(off, B)], srcs[sl])
                pltpu.sync_copy(dst_h.at[pl.ds(off, B)], dsts[sl])
                if n_chunks > 1:
                    def sidx(j, _):
                        w = pl.ds(j * 16, 16)
                        srcs[sl][w] = srcs[sl][w] * n_chunks + q
                        return 0
                    lax.fori_loop(0, B // 16, sidx, 0)
                pltpu.async_copy(tab_h.at[srcs[sl]], hs[sl], sems[sl])
                pltpu.sync_copy(al_h.at[pl.ds(off, B)], als[sl])

            def process(sl):
                pltpu.make_async_copy(tab_h.at[srcs[sl]], hs[sl],
                                      sems[sl]).wait()
                h_v, al_v = hs[sl], als[sl]

                def comp(j4, _):
                    for u in range(4):
                        j = j4 * 4 + u
                        jj = jnp.full((16,), j, jnp.int32)
                        if n_chunks > 1:
                            m0 = plsc.load_gather(
                                al_v, [jj, jnp.full((16,), 2 * q, jnp.int32)])
                            m1 = plsc.load_gather(
                                al_v,
                                [jj, jnp.full((16,), 2 * q + 1, jnp.int32)])
                        else:
                            m0 = plsc.load_gather(
                                al_v, [jj, jnp.full((16,), 0, jnp.int32)])
                            m1 = m0
                        for fb in range(F // 16):
                            w = pl.ds(fb * 16, 16)
                            m = m0 if fb < (F // 32) else m1
                            h_v[j, w] = h_v[j, w] * m
                    return 0
                lax.fori_loop(0, B // 4, comp, 0)
                pltpu.sync_copy(h_v, out_sh.at[dsts[sl]], add=True)

            prefetch(0, 0)

            def pair(p, _):
                prefetch(2 * p + 1, 1)
                process(0)

                @pl.when(2 * p + 2 < niter)
                def _():
                    prefetch(2 * p + 2, 0)
                process(1)
                return 0
            lax.fori_loop(0, niter // 2, pair, 0)
            if niter % 2 == 1:
                process(0)
            plsc.subcore_barrier()
            ob = (q if not half else c) * N + s * NPT
            pltpu.sync_copy(out_sh.at[pl.ds(s * NPT, NPT)],
                            out_h.at[pl.ds(ob, NPT)])

    return k(table, alpha, src, dst)


def kernel(x, edge_index, W1, a1s, a1d, b1, W2, a2s, a2d, b2, Wfc, bfc):
    src = edge_index[0]
    dst = edge_index[1]

    eye8 = jnp.eye(HEADS, dtype=jnp.float32)
    As = jnp.reshape(eye8[:, None, :] * a1s[:, :, None], (HEADS * HID, HEADS))
    Ad = jnp.reshape(eye8[:, None, :] * a1d[:, :, None], (HEADS * HID, HEADS))
    asd = jnp.concatenate([As, As, Ad, Ad], axis=1)

    h1, as16, ad16 = _proj1(x, W1, asd)

    alpha1 = _attn_sc(src, dst, as16, ad16)
    out1f = _msg_sc(h1.reshape(4 * N, 128), alpha1, src, dst, 4, 128, 160)
    o1c = out1f.reshape(4, N, 128)

    a2comb = jnp.concatenate(
        [jnp.repeat(a2s.T, 16, axis=1), jnp.repeat(a2d.T, 16, axis=1)], axis=1
    )
    h2, as16_2, ad16_2 = _mid(o1c, b1[None, :], W2.reshape(4, 128, HID), a2comb)

    alpha2 = _attn_sc(src, dst, as16_2, ad16_2)
    out2f = _msg_sc(h2, alpha2, src, dst, 1, HID, 200)
    o2 = out2f.reshape(2, N, HID)

    wfc_p = jnp.zeros((HID, 128), jnp.float32).at[:, :2].set(Wfc)
    bfc_p = jnp.zeros((1, 128), jnp.float32).at[0, :2].set(bfc)
    y = _fin(o2[0], o2[1], b2[None, :], wfc_p, bfc_p)
    return y[:, :2]


# speedup vs baseline: 32.3569x; 1.2431x over previous
import functools

import jax
import jax.numpy as jnp
from jax import lax
from jax.experimental import pallas as pl
from jax.experimental.pallas import tpu as pltpu
from jax.experimental.pallas import tpu_sc as plsc

N = 10000
E = 320000
D_IN = 128
HID = 64
HEADS = 8
RB = 1000


def _proj1_body(x_ref, w_ref, asd_ref, h_ref, as_ref, ad_ref):
    h = jnp.dot(x_ref[...], w_ref[...], preferred_element_type=jnp.float32)
    h_ref[...] = h
    s = jnp.dot(h, asd_ref[...], preferred_element_type=jnp.float32)
    as_ref[...] = s[:, :16]
    ad_ref[...] = s[:, 16:]


def _proj1(x, W1, asd):
    return pl.pallas_call(
        _proj1_body,
        grid=(N // RB,),
        in_specs=[
            pl.BlockSpec((RB, D_IN), lambda i: (i, 0)),
            pl.BlockSpec((D_IN, HEADS * HID), lambda i: (0, 0)),
            pl.BlockSpec((HEADS * HID, 32), lambda i: (0, 0)),
        ],
        out_specs=[
            pl.BlockSpec((RB, HEADS * HID), lambda i: (i, 0)),
            pl.BlockSpec((RB, 16), lambda i: (i, 0)),
            pl.BlockSpec((RB, 16), lambda i: (i, 0)),
        ],
        out_shape=[
            jax.ShapeDtypeStruct((N, HEADS * HID), jnp.float32),
            jax.ShapeDtypeStruct((N, 16), jnp.float32),
            jax.ShapeDtypeStruct((N, 16), jnp.float32),
        ],
    )(x, W1, asd)


def _elu(v):
    return jnp.where(v > 0, v, jnp.exp(jnp.minimum(v, 0.0)) - 1.0)


def _mid_body(o1_ref, b1_ref, w2_ref, a2_ref, h2_ref, as_ref, ad_ref):
    acc = jnp.zeros((RB, HID), jnp.float32)
    for c in range(4):
        xc = _elu(o1_ref[c] + b1_ref[0, c*128:(c+1)*128][None, :])
        acc = acc + jnp.dot(xc, w2_ref[c], preferred_element_type=jnp.float32)
    h2_ref[...] = acc
    s = jnp.dot(acc, a2_ref[...], preferred_element_type=jnp.float32)
    as_ref[...] = s[:, :16]
    ad_ref[...] = s[:, 16:]


def _mid(o1c, b1, W2c, a2comb):
    return pl.pallas_call(
        _mid_body,
        grid=(N // RB,),
        in_specs=[
            pl.BlockSpec((4, RB, 128), lambda i: (0, i, 0)),
            pl.BlockSpec((1, HEADS * HID), lambda i: (0, 0)),
            pl.BlockSpec((4, 128, HID), lambda i: (0, 0, 0)),
            pl.BlockSpec((HID, 32), lambda i: (0, 0)),
        ],
        out_specs=[
            pl.BlockSpec((RB, HID), lambda i: (i, 0)),
            pl.BlockSpec((RB, 16), lambda i: (i, 0)),
            pl.BlockSpec((RB, 16), lambda i: (i, 0)),
        ],
        out_shape=[
            jax.ShapeDtypeStruct((N, HID), jnp.float32),
            jax.ShapeDtypeStruct((N, 16), jnp.float32),
            jax.ShapeDtypeStruct((N, 16), jnp.float32),
        ],
    )(o1c, b1, W2c, a2comb)


def _fin_body(oa_ref, ob_ref, b2_ref, wfc_ref, bfc_ref, y_ref):
    t = _elu(oa_ref[...] + ob_ref[...] + b2_ref[...])
    y_ref[...] = jnp.dot(t, wfc_ref[...], preferred_element_type=jnp.float32) + bfc_ref[...]


def _fin(oa, ob, b2, wfc_p, bfc_p):
    return pl.pallas_call(
        _fin_body,
        grid=(N // RB,),
        in_specs=[
            pl.BlockSpec((RB, HID), lambda i: (i, 0)),
            pl.BlockSpec((RB, HID), lambda i: (i, 0)),
            pl.BlockSpec((1, HID), lambda i: (0, 0)),
            pl.BlockSpec((HID, 128), lambda i: (0, 0)),
            pl.BlockSpec((1, 128), lambda i: (0, 0)),
        ],
        out_specs=pl.BlockSpec((RB, 128), lambda i: (i, 0)),
        out_shape=jax.ShapeDtypeStruct((N, 128), jnp.float32),
    )(oa, ob, b2, wfc_p, bfc_p)


SC_CORES = 2
SC_TILES = 16
NPT = N // SC_TILES


def _attn_sc(src, dst, as16, ad16):
    B = 1000
    EPT1 = E // SC_TILES
    EPT2 = E // (SC_CORES * SC_TILES)
    HALF = E // SC_CORES
    mesh = plsc.VectorSubcoreMesh(core_axis_name="c", subcore_axis_name="s")

    @functools.partial(
        pl.kernel,
        out_type=jax.ShapeDtypeStruct((E, 16), jnp.float32),
        mesh=mesh,
        scratch_types=[
            pltpu.VMEM((B,), jnp.int32),
            pltpu.VMEM((B,), jnp.int32),
            pltpu.VMEM((B, 16), jnp.float32),
            pltpu.VMEM((B, 16), jnp.float32),
            pltpu.VMEM((B, 16), jnp.float32),
            pltpu.VMEM((NPT, 16), jnp.float32),
            pltpu.VMEM_SHARED((N, 16), jnp.float32),
            pltpu.SemaphoreType.DMA,
        ],
        compiler_params=pltpu.CompilerParams(use_tc_tiling_on_sc=False),
    )
    def k(src_h, dst_h, as_h, ad_h, alpha_h, src_v, dst_v, as_v, ad_v, ex_v,
          z_v, den_sh, sem):
        c = lax.axis_index("c")
        s = lax.axis_index("s")

        @plsc.parallel_loop(0, NPT, step=1, unroll=8)
        def zrow(j):
            z_v[j, :] = jnp.zeros((16,), jnp.float32)
        pltpu.sync_copy(z_v, den_sh.at[pl.ds(s * NPT, NPT)])
        plsc.subcore_barrier()

        def p1(i, _):
            off = s * EPT1 + i * B
            pltpu.sync_copy(src_h.at[pl.ds(off, B)], src_v)
            pltpu.sync_copy(dst_h.at[pl.ds(off, B)], dst_v)
            pltpu.async_copy(as_h.at[src_v], as_v, sem).wait()
            pltpu.async_copy(ad_h.at[dst_v], ad_v, sem).wait()

            @plsc.parallel_loop(0, B, step=1, unroll=8)
            def comp(j):
                e = as_v[j, :] + ad_v[j, :]
                e = jnp.maximum(e, 0.2 * e)
                ex_v[j, :] = jnp.exp(e)
            pltpu.sync_copy(ex_v, den_sh.at[dst_v], add=True)

            @pl.when(jnp.logical_and(off >= c * HALF, off < (c + 1) * HALF))
            def _():
                pltpu.sync_copy(ex_v, alpha_h.at[pl.ds(off, B)])
            return 0
        lax.fori_loop(0, EPT1 // B, p1, 0)
        plsc.subcore_barrier()

        def p2(i, _):
            off = (c * SC_TILES + s) * EPT2 + i * B
            pltpu.sync_copy(dst_h.at[pl.ds(off, B)], dst_v)
            pltpu.sync_copy(alpha_h.at[pl.ds(off, B)], ex_v)
            pltpu.async_copy(den_sh.at[dst_v], ad_v, sem).wait()

            @plsc.parallel_loop(0, B, step=1, unroll=8)
            def comp(j):
                ex_v[j, :] = ex_v[j, :] / (ad_v[j, :] + 1e-16)
            pltpu.sync_copy(ex_v, alpha_h.at[pl.ds(off, B)])
            return 0
        lax.fori_loop(0, EPT2 // B, p2, 0)

    return k(src, dst, as16, ad16)


def _msg_sc(table, alpha, src, dst, n_chunks, F, B):
    half = n_chunks == 1
    ch_per_sc = n_chunks // SC_CORES if not half else 1
    out_rows = n_chunks if not half else SC_CORES
    ept = E // (SC_CORES * SC_TILES) if half else E // SC_TILES
    mesh = plsc.VectorSubcoreMesh(core_axis_name="c", subcore_axis_name="s")

    niter = ept // B
    assert niter * B == ept and B % 4 == 0
    assert n_chunks == 1 or B % 16 == 0

    @functools.partial(
        pl.kernel,
        out_type=jax.ShapeDtypeStruct((out_rows * N, F), jnp.float32),
        mesh=mesh,
        scratch_types=[
            pltpu.VMEM((B,), jnp.int32),
            pltpu.VMEM((B,), jnp.int32),
            pltpu.VMEM((B,), jnp.int32),
            pltpu.VMEM((B,), jnp.int32),
            pltpu.VMEM((B, 16), jnp.float32),
            pltpu.VMEM((B, 16), jnp.float32),
            pltpu.VMEM((B, F), jnp.float32),
            pltpu.VMEM((B, F), jnp.float32),
            pltpu.VMEM_SHARED((N, F), jnp.float32),
            pltpu.SemaphoreType.DMA,
            pltpu.SemaphoreType.DMA,
        ],
        compiler_params=pltpu.CompilerParams(
            use_tc_tiling_on_sc=False, needs_layout_passes=False),
    )
    def k(tab_h, al_h, src_h, dst_h, out_h, src0, src1, dst0, dst1, al0, al1,
          h0, h1, out_sh, sem0, sem1):
        c = lax.axis_index("c")
        s = lax.axis_index("s")
        srcs, dsts, als, hs, sems = (src0, src1), (dst0, dst1), (al0, al1), \
            (h0, h1), (sem0, sem1)

        for q_l in range(ch_per_sc):
            q = c * ch_per_sc + q_l
            base = (c * SC_TILES + s) * ept if half else s * ept

            @plsc.parallel_loop(0, 125, step=1, unroll=8)
            def zrow(j):
                for fb in range(F // 16):
                    h0[j, pl.ds(fb * 16, 16)] = jnp.zeros((16,), jnp.float32)
            for r in range(NPT // 125):
                pltpu.sync_copy(h0.at[pl.ds(0, 125)],
                                out_sh.at[pl.ds(s * NPT + r * 125, 125)])
            plsc.subcore_barrier()

            def prefetch(i, sl):
                off = base + i * B
                pltpu.sync_copy(src_h.at[pl.ds(off, B)], srcs[sl])
                pltpu.sync_copy(dst_h.at[pl.ds(off, B)], dsts[sl])
                if n_chunks > 1:
                    @plsc.parallel_loop(0, B // 16, step=1, unroll=4)
                    def sidx(j):
                        w = pl.ds(j * 16, 16)
                        srcs[sl][w] = srcs[sl][w] * n_chunks + q
                pltpu.async_copy(tab_h.at[srcs[sl]], hs[sl], sems[sl])
                pltpu.sync_copy(al_h.at[pl.ds(off, B)], als[sl])

            def process(sl):
                pltpu.make_async_copy(tab_h.at[srcs[sl]], hs[sl],
                                      sems[sl]).wait()
                h_v, al_v = hs[sl], als[sl]

                @plsc.parallel_loop(0, B, step=1, unroll=8)
                def comp(j):
                    jj = jnp.full((16,), j, jnp.int32)
                    if n_chunks > 1:
                        m0 = plsc.load_gather(
                            al_v, [jj, jnp.full((16,), 2 * q, jnp.int32)])
                        m1 = plsc.load_gather(
                            al_v, [jj, jnp.full((16,), 2 * q + 1, jnp.int32)])
                    else:
                        m0 = plsc.load_gather(
                            al_v, [jj, jnp.full((16,), 0, jnp.int32)])
                        m1 = m0
                    for fb in range(F // 16):
                        w = pl.ds(fb * 16, 16)
                        m = m0 if fb < (F // 32) else m1
                        h_v[j, w] = h_v[j, w] * m
                pltpu.sync_copy(h_v, out_sh.at[dsts[sl]], add=True)

            prefetch(0, 0)

            def pair(p, _):
                prefetch(2 * p + 1, 1)
                process(0)

                @pl.when(2 * p + 2 < niter)
                def _():
                    prefetch(2 * p + 2, 0)
                process(1)
                return 0
            lax.fori_loop(0, niter // 2, pair, 0)
            if niter % 2 == 1:
                process(0)
            plsc.subcore_barrier()
            ob = (q if not half else c) * N + s * NPT
            pltpu.sync_copy(out_sh.at[pl.ds(s * NPT, NPT)],
                            out_h.at[pl.ds(ob, NPT)])

    return k(table, alpha, src, dst)


def kernel(x, edge_index, W1, a1s, a1d, b1, W2, a2s, a2d, b2, Wfc, bfc):
    src = edge_index[0]
    dst = edge_index[1]

    eye8 = jnp.eye(HEADS, dtype=jnp.float32)
    As = jnp.reshape(eye8[:, None, :] * a1s[:, :, None], (HEADS * HID, HEADS))
    Ad = jnp.reshape(eye8[:, None, :] * a1d[:, :, None], (HEADS * HID, HEADS))
    asd = jnp.concatenate([As, As, Ad, Ad], axis=1)

    h1, as16, ad16 = _proj1(x, W1, asd)

    alpha1 = _attn_sc(src, dst, as16, ad16)
    out1f = _msg_sc(h1.reshape(4 * N, 128), alpha1, src, dst, 4, 128, 160)
    o1c = out1f.reshape(4, N, 128)

    a2comb = jnp.concatenate(
        [jnp.repeat(a2s.T, 16, axis=1), jnp.repeat(a2d.T, 16, axis=1)], axis=1
    )
    h2, as16_2, ad16_2 = _mid(o1c, b1[None, :], W2.reshape(4, 128, HID), a2comb)

    alpha2 = _attn_sc(src, dst, as16_2, ad16_2)
    out2f = _msg_sc(h2, alpha2, src, dst, 1, HID, 200)
    o2 = out2f.reshape(2, N, HID)

    wfc_p = jnp.zeros((HID, 128), jnp.float32).at[:, :2].set(Wfc)
    bfc_p = jnp.zeros((1, 128), jnp.float32).at[0, :2].set(bfc)
    y = _fin(o2[0], o2[1], b2[None, :], wfc_p, bfc_p)
    return y[:, :2]
